# Initial kernel scaffold; baseline (speedup 1.0000x reference)
import jax
import jax.numpy as jnp
from jax.experimental import pallas as pl


def kernel(x, conv_w, gcn_w, gcn_b):
    raise NotImplementedError("write your pallas kernel here")


# speedup vs baseline: 4.3052x; 4.3052x over previous
import jax
import jax.numpy as jnp
from jax.experimental import pallas as pl

_PATCH = 2
_EMBED = 96
_G = 256
_RB = 32
_NT = _G // _RB
_K = 8

_HIGH = jax.lax.Precision.HIGHEST


def _gcn_kernel(p_top, p_mid, p_bot, wkt, gw, bias, out_ref):
    bi = pl.program_id(0)
    ti = pl.program_id(1)

    cw = jnp.dot(wkt[...], gw[...], preferred_element_type=jnp.float32,
                 precision=_HIGH)
    msg = jnp.dot(p_mid[0].reshape(_RB * _G, _K), cw,
                  preferred_element_type=jnp.float32, precision=_HIGH)
    b2 = bias[...]

    @pl.when(bi != 0)
    def _plain():
        out_ref[0] = msg + b2

    @pl.when(bi == 0)
    def _stencil():
        row0 = ti * _RB
        idx = jax.lax.broadcasted_iota(jnp.int32, (_RB * _G, 1), 0)
        ii = row0 + (idx >> 8)
        jj = idx & (_G - 1)
        deg = (1.0
               + (ii > 0).astype(jnp.float32)
               + (ii < _G - 1).astype(jnp.float32)
               + (jj > 0).astype(jnp.float32)
               + (jj < _G - 1).astype(jnp.float32))
        dinv = jax.lax.rsqrt(deg)
        y = msg * dinv

        jt = jax.lax.broadcasted_iota(jnp.int32, (_G, 1), 0)
        jmask = ((jt > 0).astype(jnp.float32)
                 + (jt < _G - 1).astype(jnp.float32))
        deg_t = (1.0 + ((row0 - 1) > 0).astype(jnp.float32)
                 + ((row0 - 1) < _G - 1).astype(jnp.float32) + jmask)
        deg_b = (1.0 + ((row0 + _RB) > 0).astype(jnp.float32)
                 + ((row0 + _RB) < _G - 1).astype(jnp.float32) + jmask)
        y_top = (jnp.dot(p_top[0, _RB - 1], cw,
                         preferred_element_type=jnp.float32, precision=_HIGH)
                 * jax.lax.rsqrt(deg_t))
        y_bot = (jnp.dot(p_bot[0, 0], cw,
                         preferred_element_type=jnp.float32, precision=_HIGH)
                 * jax.lax.rsqrt(deg_b))

        up = jnp.concatenate([y_top, y[:-_G]], axis=0)
        down = jnp.concatenate([y[_G:], y_bot], axis=0)
        zrow = jnp.zeros((1, _EMBED), jnp.float32)
        left = jnp.concatenate([zrow, y[:-1]], axis=0)
        right = jnp.concatenate([y[1:], zrow], axis=0)
        lm = (jj > 0).astype(jnp.float32)
        rm = (jj < _G - 1).astype(jnp.float32)
        acc = y + up + down + left * lm + right * rm
        out_ref[0] = acc * dinv + b2


def kernel(x, conv_w, gcn_w, gcn_b):
    b, cin, hh, ww = x.shape
    h, w = hh // _PATCH, ww // _PATCH
    n = h * w
    kdim = cin * _PATCH * _PATCH

    patches = (x.reshape(b, cin, h, _PATCH, w, _PATCH)
               .transpose(0, 2, 4, 1, 3, 5)
               .reshape(b, h, w, kdim))
    patches = jnp.pad(patches, ((0, 0), (_RB, _RB), (0, 0), (0, _K - kdim)))
    wkt = jnp.pad(conv_w.reshape(_EMBED, kdim).T, ((0, _K - kdim), (0, 0)))
    bias2 = gcn_b.reshape(1, _EMBED)

    out = pl.pallas_call(
        _gcn_kernel,
        grid=(b, _NT),
        in_specs=[
            pl.BlockSpec((1, _RB, _G, _K), lambda bi, ti: (bi, ti, 0, 0)),
            pl.BlockSpec((1, _RB, _G, _K), lambda bi, ti: (bi, ti + 1, 0, 0)),
            pl.BlockSpec((1, _RB, _G, _K), lambda bi, ti: (bi, ti + 2, 0, 0)),
            pl.BlockSpec((_K, _EMBED), lambda bi, ti: (0, 0)),
            pl.BlockSpec((_EMBED, _EMBED), lambda bi, ti: (0, 0)),
            pl.BlockSpec((1, _EMBED), lambda bi, ti: (0, 0)),
        ],
        out_specs=pl.BlockSpec((1, _RB * _G, _EMBED), lambda bi, ti: (bi, ti, 0)),
        out_shape=jax.ShapeDtypeStruct((b, n, _EMBED), jnp.float32),
    )(patches, patches, patches, wkt, gcn_w, bias2)
    return out


# speedup vs baseline: 7.5951x; 1.7642x over previous
import jax
import jax.numpy as jnp
from jax.experimental import pallas as pl

_PATCH = 2
_EMBED = 96
_G = 256
_RB = 32
_NT = _G // _RB
_KD = 4

def _slab_dot(cf, p):
    return jax.lax.dot_general(cf, p, (((1,), (0,)), ((), ())),
                               preferred_element_type=jnp.float32)


def _gcn_kernel(p_top, p_mid, p_bot, wk, gw, bias, out_ref):
    bi = pl.program_id(0)
    ti = pl.program_id(1)

    cf = jax.lax.dot_general(gw[...], wk[...], (((0,), (0,)), ((), ())),
                             preferred_element_type=jnp.float32,
                             precision=jax.lax.Precision.HIGHEST)
    pm = p_mid[0]
    b2 = bias[...]

    @pl.when(bi != 0)
    def _plain():
        for i in range(_RB):
            ms = _slab_dot(cf, pm[:, i, :])
            out_ref[0, i * _G:(i + 1) * _G, :] = ms.T + b2

    @pl.when(bi == 0)
    def _stencil():
        row0 = ti * _RB
        jt = jax.lax.broadcasted_iota(jnp.int32, (1, _G), 1)
        jdeg = (jt > 0).astype(jnp.float32) + (jt < _G - 1).astype(jnp.float32)
        d_int = jax.lax.rsqrt(3.0 + jdeg)
        d_edge = jax.lax.rsqrt(2.0 + jdeg)

        ys = [_slab_dot(cf, p_top[0, :, _RB - 1, :]) * d_int]
        for i in range(_RB):
            gi = row0 + i
            is_edge = jnp.logical_or(gi == 0, gi == _G - 1)
            drow = jnp.where(is_edge, d_edge, d_int)
            ys.append(_slab_dot(cf, pm[:, i, :]) * drow)
        ys.append(_slab_dot(cf, p_bot[0, :, 0, :]) * d_int)

        zcol = jnp.zeros((_EMBED, 1), jnp.float32)
        for i in range(_RB):
            gi = row0 + i
            is_edge = jnp.logical_or(gi == 0, gi == _G - 1)
            drow = jnp.where(is_edge, d_edge, d_int)
            y = ys[i + 1]
            left = jnp.concatenate([zcol, y[:, :-1]], axis=1)
            right = jnp.concatenate([y[:, 1:], zcol], axis=1)
            acc = ((y + ys[i]) + (ys[i + 2] + left) + right) * drow
            out_ref[0, i * _G:(i + 1) * _G, :] = acc.T + b2


def kernel(x, conv_w, gcn_w, gcn_b):
    b, cin, hh, ww = x.shape
    h, w = hh // _PATCH, ww // _PATCH
    n = h * w

    pt = (x.reshape(b, h, _PATCH, w, _PATCH)
          .transpose(0, 2, 4, 1, 3)
          .reshape(b, _KD, h, w))
    pt = jnp.pad(pt, ((0, 0), (0, 0), (_RB, _RB), (0, 0)))
    wk = conv_w.reshape(_EMBED, _KD)
    bias2 = gcn_b.reshape(1, _EMBED)

    out = pl.pallas_call(
        _gcn_kernel,
        grid=(b, _NT),
        in_specs=[
            pl.BlockSpec((1, _KD, _RB, _G), lambda bi, ti: (bi, 0, ti, 0)),
            pl.BlockSpec((1, _KD, _RB, _G), lambda bi, ti: (bi, 0, ti + 1, 0)),
            pl.BlockSpec((1, _KD, _RB, _G), lambda bi, ti: (bi, 0, ti + 2, 0)),
            pl.BlockSpec((_EMBED, _KD), lambda bi, ti: (0, 0)),
            pl.BlockSpec((_EMBED, _EMBED), lambda bi, ti: (0, 0)),
            pl.BlockSpec((1, _EMBED), lambda bi, ti: (0, 0)),
        ],
        out_specs=pl.BlockSpec((1, _RB * _G, _EMBED), lambda bi, ti: (bi, ti, 0)),
        out_shape=jax.ShapeDtypeStruct((b, n, _EMBED), jnp.float32),
    )(pt, pt, pt, wk, gcn_w, bias2)
    return out


# speedup vs baseline: 7.9987x; 1.0531x over previous
import jax
import jax.numpy as jnp
from jax.experimental import pallas as pl

_PATCH = 2
_EMBED = 96
_G = 256
_RB = 32
_NT = _G // _RB
_KD = 4


def _slab_dot(cf, p):
    return jax.lax.dot_general(cf, p, (((1,), (0,)), ((), ())),
                               preferred_element_type=jnp.float32)


def _slab(ev, od, r):
    return jnp.concatenate([ev[r:r + 1], od[r:r + 1],
                            ev[r + 1:r + 2], od[r + 1:r + 2]], axis=0)


def _gcn_kernel(te, to, me, mo, be, bo, wk, gw, bias, out_ref):
    bi = pl.program_id(0)
    ti = pl.program_id(1)

    cf = jax.lax.dot_general(gw[...], wk[...], (((0,), (0,)), ((), ())),
                             preferred_element_type=jnp.float32,
                             precision=jax.lax.Precision.HIGHEST)
    xe, xo = me[0], mo[0]
    b2 = bias[...]

    @pl.when(bi != 0)
    def _plain():
        for i in range(_RB):
            ms = _slab_dot(cf, _slab(xe, xo, 2 * i))
            out_ref[0, i * _G:(i + 1) * _G, :] = ms.T + b2

    @pl.when(bi == 0)
    def _stencil():
        row0 = ti * _RB
        jt = jax.lax.broadcasted_iota(jnp.int32, (1, _G), 1)
        jdeg = (jt > 0).astype(jnp.float32) + (jt < _G - 1).astype(jnp.float32)
        d_int = jax.lax.rsqrt(3.0 + jdeg)
        d_edge = jax.lax.rsqrt(2.0 + jdeg)

        tmask = (ti > 0).astype(jnp.float32)
        bmask = (ti < _NT - 1).astype(jnp.float32)
        ys = [_slab_dot(cf, _slab(te[0, 0], to[0, 0], 0)) * (d_int * tmask)]
        for i in range(_RB):
            gi = row0 + i
            is_edge = jnp.logical_or(gi == 0, gi == _G - 1)
            drow = jnp.where(is_edge, d_edge, d_int)
            ys.append(_slab_dot(cf, _slab(xe, xo, 2 * i)) * drow)
        ys.append(_slab_dot(cf, _slab(be[0, 0], bo[0, 0], 0)) * (d_int * bmask))

        zcol = jnp.zeros((_EMBED, 1), jnp.float32)
        for i in range(_RB):
            gi = row0 + i
            is_edge = jnp.logical_or(gi == 0, gi == _G - 1)
            drow = jnp.where(is_edge, d_edge, d_int)
            y = ys[i + 1]
            left = jnp.concatenate([zcol, y[:, :-1]], axis=1)
            right = jnp.concatenate([y[:, 1:], zcol], axis=1)
            acc = ((y + ys[i]) + (ys[i + 2] + left) + right) * drow
            out_ref[0, i * _G:(i + 1) * _G, :] = acc.T + b2


def kernel(x, conv_w, gcn_w, gcn_b):
    b, cin, hh, ww = x.shape
    h, w = hh // _PATCH, ww // _PATCH
    n = h * w

    xe = x[:, 0, :, 0::2]
    xo = x[:, 0, :, 1::2]
    xep = xe.reshape(b, h, _PATCH, w)
    xop = xo.reshape(b, h, _PATCH, w)
    wk = conv_w.reshape(_EMBED, _KD)
    bias2 = gcn_b.reshape(1, _EMBED)

    halo_top = lambda bi, ti: (bi, jnp.maximum(ti * _RB - 1, 0), 0, 0)
    halo_bot = lambda bi, ti: (bi, jnp.minimum(ti * _RB + _RB, h - 1), 0, 0)
    out = pl.pallas_call(
        _gcn_kernel,
        grid=(b, _NT),
        in_specs=[
            pl.BlockSpec((1, 1, _PATCH, w), halo_top),
            pl.BlockSpec((1, 1, _PATCH, w), halo_top),
            pl.BlockSpec((1, _PATCH * _RB, w), lambda bi, ti: (bi, ti, 0)),
            pl.BlockSpec((1, _PATCH * _RB, w), lambda bi, ti: (bi, ti, 0)),
            pl.BlockSpec((1, 1, _PATCH, w), halo_bot),
            pl.BlockSpec((1, 1, _PATCH, w), halo_bot),
            pl.BlockSpec((_EMBED, _KD), lambda bi, ti: (0, 0)),
            pl.BlockSpec((_EMBED, _EMBED), lambda bi, ti: (0, 0)),
            pl.BlockSpec((1, _EMBED), lambda bi, ti: (0, 0)),
        ],
        out_specs=pl.BlockSpec((1, _RB * _G, _EMBED), lambda bi, ti: (bi, ti, 0)),
        out_shape=jax.ShapeDtypeStruct((b, n, _EMBED), jnp.float32),
    )(xep, xop, xe, xo, xep, xop, wk, gcn_w, bias2)
    return out


# speedup vs baseline: 14.7310x; 1.8417x over previous
import jax
import jax.numpy as jnp
from jax.experimental import pallas as pl

_PATCH = 2
_EMBED = 96
_G = 256
_RB = 32
_NT = _G // _RB
_KD = 4


def _slab_dot(cf, p):
    return jax.lax.dot_general(cf, p, (((1,), (0,)), ((), ())),
                               preferred_element_type=jnp.float32)


def _slab(ev, od, r):
    return jnp.concatenate([ev[r:r + 1], od[r:r + 1],
                            ev[r + 1:r + 2], od[r + 1:r + 2]], axis=0)


def _deint(xm):
    r = xm.shape[0]
    idx = jax.lax.broadcasted_iota(jnp.int32, (r, 64), 1) * 2
    pe, po = [], []
    for c in range(4):
        ch = xm[:, 128 * c:128 * (c + 1)]
        pe.append(jnp.take_along_axis(ch, idx, axis=1))
        po.append(jnp.take_along_axis(ch, idx + 1, axis=1))
    return jnp.concatenate(pe, axis=1), jnp.concatenate(po, axis=1)


def _gcn_kernel(x_top, x_mid, x_bot, wk, gw, bias, out_ref):
    bi = pl.program_id(0)
    ti = pl.program_id(1)

    cf = jax.lax.dot_general(gw[...], wk[...], (((0,), (0,)), ((), ())),
                             preferred_element_type=jnp.float32,
                             precision=jax.lax.Precision.HIGHEST)
    xe, xo = _deint(x_mid[0])
    b2 = bias[...]

    @pl.when(bi != 0)
    def _plain():
        for i in range(_RB):
            ms = _slab_dot(cf, _slab(xe, xo, 2 * i))
            out_ref[0, i * _G:(i + 1) * _G, :] = ms.T + b2

    @pl.when(bi == 0)
    def _stencil():
        row0 = ti * _RB
        jt = jax.lax.broadcasted_iota(jnp.int32, (1, _G), 1)
        jdeg = (jt > 0).astype(jnp.float32) + (jt < _G - 1).astype(jnp.float32)
        d_int = jax.lax.rsqrt(3.0 + jdeg)
        d_edge = jax.lax.rsqrt(2.0 + jdeg)

        tmask = (ti > 0).astype(jnp.float32)
        bmask = (ti < _NT - 1).astype(jnp.float32)
        t_e, t_o = _deint(x_top[0, 0])
        ys = [_slab_dot(cf, _slab(t_e, t_o, 0)) * (d_int * tmask)]
        for i in range(_RB):
            gi = row0 + i
            is_edge = jnp.logical_or(gi == 0, gi == _G - 1)
            drow = jnp.where(is_edge, d_edge, d_int)
            ys.append(_slab_dot(cf, _slab(xe, xo, 2 * i)) * drow)
        b_e, b_o = _deint(x_bot[0, 0])
        ys.append(_slab_dot(cf, _slab(b_e, b_o, 0)) * (d_int * bmask))

        zcol = jnp.zeros((_EMBED, 1), jnp.float32)
        for i in range(_RB):
            gi = row0 + i
            is_edge = jnp.logical_or(gi == 0, gi == _G - 1)
            drow = jnp.where(is_edge, d_edge, d_int)
            y = ys[i + 1]
            left = jnp.concatenate([zcol, y[:, :-1]], axis=1)
            right = jnp.concatenate([y[:, 1:], zcol], axis=1)
            acc = ((y + ys[i]) + (ys[i + 2] + left) + right) * drow
            out_ref[0, i * _G:(i + 1) * _G, :] = acc.T + b2


def kernel(x, conv_w, gcn_w, gcn_b):
    b, cin, hh, ww = x.shape
    h, w = hh // _PATCH, ww // _PATCH
    n = h * w

    xr = x.reshape(b, hh, ww)
    xp = x.reshape(b, h, _PATCH, ww)
    wk = conv_w.reshape(_EMBED, _KD)
    bias2 = gcn_b.reshape(1, _EMBED)

    out = pl.pallas_call(
        _gcn_kernel,
        grid=(b, _NT),
        in_specs=[
            pl.BlockSpec((1, 1, _PATCH, ww),
                         lambda bi, ti: (bi, jnp.maximum(ti * _RB - 1, 0), 0, 0)),
            pl.BlockSpec((1, _PATCH * _RB, ww), lambda bi, ti: (bi, ti, 0)),
            pl.BlockSpec((1, 1, _PATCH, ww),
                         lambda bi, ti: (bi, jnp.minimum(ti * _RB + _RB, h - 1), 0, 0)),
            pl.BlockSpec((_EMBED, _KD), lambda bi, ti: (0, 0)),
            pl.BlockSpec((_EMBED, _EMBED), lambda bi, ti: (0, 0)),
            pl.BlockSpec((1, _EMBED), lambda bi, ti: (0, 0)),
        ],
        out_specs=pl.BlockSpec((1, _RB * _G, _EMBED), lambda bi, ti: (bi, ti, 0)),
        out_shape=jax.ShapeDtypeStruct((b, n, _EMBED), jnp.float32),
    )(xp, xr, xp, wk, gcn_w, bias2)
    return out


# speedup vs baseline: 44.1136x; 2.9946x over previous
import jax
import jax.numpy as jnp
from jax.experimental import pallas as pl

_PATCH = 2
_EMBED = 96
_G = 256
_RB = 32
_NT = _G // _RB
_KD = 4


def _slab_dot(cf, p):
    return jax.lax.dot_general(cf, p, (((1,), (0,)), ((), ())),
                               preferred_element_type=jnp.float32)


def _slab(ev, od, r):
    return jnp.concatenate([ev[r:r + 1], od[r:r + 1],
                            ev[r + 1:r + 2], od[r + 1:r + 2]], axis=0)


def _deint(xm):
    r = xm.shape[0]
    idx = jax.lax.broadcasted_iota(jnp.int32, (r, 64), 1) * 2
    pe, po = [], []
    for c in range(4):
        ch = xm[:, 128 * c:128 * (c + 1)]
        pe.append(jnp.take_along_axis(ch, idx, axis=1))
        po.append(jnp.take_along_axis(ch, idx + 1, axis=1))
    return jnp.concatenate(pe, axis=1), jnp.concatenate(po, axis=1)


def _gcn_kernel(x_top, x_mid, x_bot, wk, gw, bias, out_ref):
    bi = pl.program_id(0)
    ti = pl.program_id(1)

    cf = jax.lax.dot_general(gw[...], wk[...], (((0,), (0,)), ((), ())),
                             preferred_element_type=jnp.float32,
                             precision=jax.lax.Precision.HIGHEST)
    xe, xo = _deint(x_mid[0])
    b2 = bias[...]

    @pl.when(bi != 0)
    def _plain():
        for i in range(_RB):
            ms = _slab_dot(cf, _slab(xe, xo, 2 * i))
            out_ref[0, :, i * _G:(i + 1) * _G] = ms + b2

    @pl.when(bi == 0)
    def _stencil():
        row0 = ti * _RB
        jt = jax.lax.broadcasted_iota(jnp.int32, (1, _G), 1)
        jdeg = (jt > 0).astype(jnp.float32) + (jt < _G - 1).astype(jnp.float32)
        d_int = jax.lax.rsqrt(3.0 + jdeg)
        d_edge = jax.lax.rsqrt(2.0 + jdeg)

        tmask = (ti > 0).astype(jnp.float32)
        bmask = (ti < _NT - 1).astype(jnp.float32)
        t_e, t_o = _deint(x_top[0, 0])
        ys = [_slab_dot(cf, _slab(t_e, t_o, 0)) * (d_int * tmask)]
        for i in range(_RB):
            gi = row0 + i
            is_edge = jnp.logical_or(gi == 0, gi == _G - 1)
            drow = jnp.where(is_edge, d_edge, d_int)
            ys.append(_slab_dot(cf, _slab(xe, xo, 2 * i)) * drow)
        b_e, b_o = _deint(x_bot[0, 0])
        ys.append(_slab_dot(cf, _slab(b_e, b_o, 0)) * (d_int * bmask))

        zcol = jnp.zeros((_EMBED, 1), jnp.float32)
        for i in range(_RB):
            gi = row0 + i
            is_edge = jnp.logical_or(gi == 0, gi == _G - 1)
            drow = jnp.where(is_edge, d_edge, d_int)
            y = ys[i + 1]
            left = jnp.concatenate([zcol, y[:, :-1]], axis=1)
            right = jnp.concatenate([y[:, 1:], zcol], axis=1)
            acc = ((y + ys[i]) + (ys[i + 2] + left) + right) * drow
            out_ref[0, :, i * _G:(i + 1) * _G] = acc + b2


def kernel(x, conv_w, gcn_w, gcn_b):
    b, cin, hh, ww = x.shape
    h, w = hh // _PATCH, ww // _PATCH
    n = h * w

    xr = x.reshape(b, hh, ww)
    xp = x.reshape(b, h, _PATCH, ww)
    wk = conv_w.reshape(_EMBED, _KD)
    bias2 = gcn_b.reshape(_EMBED, 1)

    out = pl.pallas_call(
        _gcn_kernel,
        grid=(b, _NT),
        in_specs=[
            pl.BlockSpec((1, 1, _PATCH, ww),
                         lambda bi, ti: (bi, jnp.maximum(ti * _RB - 1, 0), 0, 0)),
            pl.BlockSpec((1, _PATCH * _RB, ww), lambda bi, ti: (bi, ti, 0)),
            pl.BlockSpec((1, 1, _PATCH, ww),
                         lambda bi, ti: (bi, jnp.minimum(ti * _RB + _RB, h - 1), 0, 0)),
            pl.BlockSpec((_EMBED, _KD), lambda bi, ti: (0, 0)),
            pl.BlockSpec((_EMBED, _EMBED), lambda bi, ti: (0, 0)),
            pl.BlockSpec((_EMBED, 1), lambda bi, ti: (0, 0)),
        ],
        out_specs=pl.BlockSpec((1, _EMBED, _RB * _G), lambda bi, ti: (bi, 0, ti)),
        out_shape=jax.ShapeDtypeStruct((b, _EMBED, n), jnp.float32),
    )(xp, xr, xp, wk, gcn_w, bias2)
    return jnp.transpose(out, (0, 2, 1))


# speedup vs baseline: 50.4742x; 1.1442x over previous
import jax
import jax.numpy as jnp
from jax.experimental import pallas as pl

_PATCH = 2
_EMBED = 96
_G = 256
_RB = 64
_NT = _G // _RB
_KD = 4


def _slab_dot(cf, p):
    return jax.lax.dot_general(cf, p, (((1,), (0,)), ((), ())),
                               preferred_element_type=jnp.float32)


def _slab(ev, od, r):
    return jnp.concatenate([ev[r:r + 1], od[r:r + 1],
                            ev[r + 1:r + 2], od[r + 1:r + 2]], axis=0)


def _deint(xm):
    r = xm.shape[0]
    idx = jax.lax.broadcasted_iota(jnp.int32, (r, 64), 1) * 2
    pe, po = [], []
    for c in range(4):
        ch = xm[:, 128 * c:128 * (c + 1)]
        pe.append(jnp.take_along_axis(ch, idx, axis=1))
        po.append(jnp.take_along_axis(ch, idx + 1, axis=1))
    return jnp.concatenate(pe, axis=1), jnp.concatenate(po, axis=1)


def _gcn_kernel(x_top, x_mid, x_bot, wk, gw, bias, out_ref):
    bi = pl.program_id(0)
    ti = pl.program_id(1)

    cf = jax.lax.dot_general(gw[...], wk[...], (((0,), (0,)), ((), ())),
                             preferred_element_type=jnp.float32,
                             precision=jax.lax.Precision.HIGHEST)
    xe, xo = _deint(x_mid[0])
    b2 = bias[...]

    @pl.when(bi != 0)
    def _plain():
        for i in range(_RB):
            ms = _slab_dot(cf, _slab(xe, xo, 2 * i))
            out_ref[0, :, i * _G:(i + 1) * _G] = ms + b2

    @pl.when(bi == 0)
    def _stencil():
        row0 = ti * _RB
        jt = jax.lax.broadcasted_iota(jnp.int32, (1, _G), 1)
        jdeg = (jt > 0).astype(jnp.float32) + (jt < _G - 1).astype(jnp.float32)
        d_int = jax.lax.rsqrt(3.0 + jdeg)
        d_edge = jax.lax.rsqrt(2.0 + jdeg)

        tmask = (ti > 0).astype(jnp.float32)
        bmask = (ti < _NT - 1).astype(jnp.float32)
        t_e, t_o = _deint(x_top[0, 0])
        ys = [_slab_dot(cf, _slab(t_e, t_o, 0)) * (d_int * tmask)]
        for i in range(_RB):
            gi = row0 + i
            is_edge = jnp.logical_or(gi == 0, gi == _G - 1)
            drow = jnp.where(is_edge, d_edge, d_int)
            ys.append(_slab_dot(cf, _slab(xe, xo, 2 * i)) * drow)
        b_e, b_o = _deint(x_bot[0, 0])
        ys.append(_slab_dot(cf, _slab(b_e, b_o, 0)) * (d_int * bmask))

        zcol = jnp.zeros((_EMBED, 1), jnp.float32)
        for i in range(_RB):
            gi = row0 + i
            is_edge = jnp.logical_or(gi == 0, gi == _G - 1)
            drow = jnp.where(is_edge, d_edge, d_int)
            y = ys[i + 1]
            left = jnp.concatenate([zcol, y[:, :-1]], axis=1)
            right = jnp.concatenate([y[:, 1:], zcol], axis=1)
            acc = ((y + ys[i]) + (ys[i + 2] + left) + right) * drow
            out_ref[0, :, i * _G:(i + 1) * _G] = acc + b2


def kernel(x, conv_w, gcn_w, gcn_b):
    b, cin, hh, ww = x.shape
    h, w = hh // _PATCH, ww // _PATCH
    n = h * w

    xr = x.reshape(b, hh, ww)
    xp = x.reshape(b, h, _PATCH, ww)
    wk = conv_w.reshape(_EMBED, _KD)
    bias2 = gcn_b.reshape(_EMBED, 1)

    out = pl.pallas_call(
        _gcn_kernel,
        grid=(b, _NT),
        in_specs=[
            pl.BlockSpec((1, 1, _PATCH, ww),
                         lambda bi, ti: (bi, jnp.maximum(ti * _RB - 1, 0), 0, 0)),
            pl.BlockSpec((1, _PATCH * _RB, ww), lambda bi, ti: (bi, ti, 0)),
            pl.BlockSpec((1, 1, _PATCH, ww),
                         lambda bi, ti: (bi, jnp.minimum(ti * _RB + _RB, h - 1), 0, 0)),
            pl.BlockSpec((_EMBED, _KD), lambda bi, ti: (0, 0)),
            pl.BlockSpec((_EMBED, _EMBED), lambda bi, ti: (0, 0)),
            pl.BlockSpec((_EMBED, 1), lambda bi, ti: (0, 0)),
        ],
        out_specs=pl.BlockSpec((1, _EMBED, _RB * _G), lambda bi, ti: (bi, 0, ti)),
        out_shape=jax.ShapeDtypeStruct((b, _EMBED, n), jnp.float32),
    )(xp, xr, xp, wk, gcn_w, bias2)
    return jnp.transpose(out, (0, 2, 1))
